# baseline (speedup 1.0000x reference)
import jax
import jax.numpy as jnp
from jax import lax
from jax.experimental import pallas as pl
from jax.experimental.pallas import tpu as pltpu
from jax.experimental.pallas import tpu_sc as plsc

_N = 10000
_E = 320000
_D = 128
_DE = 16
_G = 256
_L = 2
_T = 2

_NC, _NS = 2, 16
_NW = _NC * _NS
_EW = _E // _NW
_CK = 80
_NCHUNK = _EW // _CK
_SPAN = 640
_SPAN_LAST = _N - (_NS - 1) * _SPAN
_NP = _NS * _SPAN


def _edge_body(src_h, dst_h, hn_h, hd_h, eb_h, w_h,
               u_out, sig_out,
               src_a, dst_a, rows_a, eb_a, hdb_a, exc_a,
               src_b, dst_b, rows_b, eb_b, hdb_b, exc_b,
               exe_v, sigbuf_v, w_l, u_sh, sig_sh,
               gsem_a, esem_a, hsem_a,
               gsem_b, esem_b, hsem_b):
    c = lax.axis_index("c")
    s = lax.axis_index("s")
    wid = c * _NS + s
    z16f = jnp.zeros((16,), jnp.float32)
    srcs = (src_a, src_b)
    dsts = (dst_a, dst_b)
    rows = (rows_a, rows_b)
    ebs = (eb_a, eb_b)
    hdbs = (hdb_a, hdb_b)
    excs = (exc_a, exc_b)
    gsems = (gsem_a, gsem_b)
    esems = (esem_a, esem_b)
    hsems = (hsem_a, hsem_b)

    @pl.loop(0, _CK)
    def _zrow(r):
        for k in range(8):
            exe_v[r, pl.ds(k * 16, 16)] = z16f
    for i in range(_SPAN // 16):
        sigbuf_v[pl.ds(i * 16, 16)] = z16f

    @pl.when(s < _NS - 1)
    def _():
        pltpu.sync_copy(sigbuf_v, sig_sh.at[pl.ds(s * _SPAN, _SPAN)])

    @pl.when(s == _NS - 1)
    def _():
        pltpu.sync_copy(sigbuf_v.at[pl.ds(0, _SPAN_LAST)],
                        sig_sh.at[pl.ds(s * _SPAN, _SPAN_LAST)])

    @pl.when(s < _NS - 1)
    def _():
        for b in range(_SPAN // _CK):
            pltpu.sync_copy(exe_v, u_sh.at[pl.ds(s * _SPAN + b * _CK, _CK), :])

    @pl.when(s == _NS - 1)
    def _():
        for b in range(_SPAN_LAST // _CK):
            pltpu.sync_copy(exe_v, u_sh.at[pl.ds(s * _SPAN + b * _CK, _CK), :])

    pltpu.sync_copy(w_h, w_l)
    plsc.subcore_barrier()

    iota16 = lax.iota(jnp.int32, 16)
    ebase = wid * _EW

    def fetch(bi, ci):
        base = ebase + ci * _CK
        pltpu.sync_copy(src_h.at[pl.ds(base, _CK)], srcs[bi])
        pltpu.sync_copy(dst_h.at[pl.ds(base, _CK)], dsts[bi])
        pltpu.async_copy(hn_h.at[srcs[bi]], rows[bi], gsems[bi])
        pltpu.async_copy(hd_h.at[dsts[bi]], hdbs[bi], hsems[bi])
        pltpu.async_copy(eb_h.at[pl.ds(base, _CK), :], ebs[bi], esems[bi])

    def wait_fetch(bi):
        pltpu.make_async_copy(hn_h.at[srcs[bi]], rows[bi], gsems[bi]).wait()
        pltpu.make_async_copy(hd_h.at[dsts[bi]], hdbs[bi], hsems[bi]).wait()
        pltpu.make_async_copy(eb_h.at[pl.ds(0, _CK), :], ebs[bi],
                              esems[bi]).wait()

    def compute(bi):
        rows_v, eb_v = rows[bi], ebs[bi]
        dst_v, exc_v, hdb_v = dsts[bi], excs[bi], hdbs[bi]

        @pl.loop(0, _CK // 16)
        def _group(j):
            r0 = j * 16
            riv = r0 + iota16
            hdj = plsc.load_gather(hdb_v, [riv])
            s = jnp.zeros((16,), jnp.float32)
            for w in range(_D // 2):
                cw = jnp.full((16,), w, jnp.int32)
                hw = plsc.load_gather(rows_v, [riv, cw])
                ew = plsc.load_gather(eb_v, [riv, cw])
                h0, h1 = plsc.unpack(plsc.bitcast(hw, jnp.bfloat16),
                                     format=plsc.PackFormat.INTERLEAVED)
                e0, e1 = plsc.unpack(plsc.bitcast(ew, jnp.bfloat16),
                                     format=plsc.PackFormat.INTERLEAVED)
                for half, (hv, ev) in enumerate(((h0, e0), (h1, e1))):
                    f = 2 * w + half
                    x = hv + ev
                    evec = jnp.where(x >= 0, x, 0.01 * x)
                    plsc.store_scatter(exe_v, [riv, jnp.full((16,), f,
                                                             jnp.int32)], evec)
                    s = s + evec * w_l[f, :]
            lg = hdj + s
            logit = jnp.where(lg >= 0, lg, 0.01 * lg)
            ex = jnp.exp(logit)
            plsc.store_scatter(exc_v, [riv], ex)
            for f in range(_D):
                cf = jnp.full((16,), f, jnp.int32)
                ev = plsc.load_gather(exe_v, [riv, cf])
                plsc.store_scatter(exe_v, [riv, cf], ev * ex)

        pltpu.sync_copy(exe_v, u_sh.at[dst_v], add=True)
        pltpu.sync_copy(exc_v, sig_sh.at[dst_v], add=True)

    fetch(0, 0)

    @pl.loop(0, (_NCHUNK - 1) // 2)
    def _pair(p):
        c0 = 2 * p
        fetch(1, c0 + 1)
        wait_fetch(0)
        compute(0)
        fetch(0, c0 + 2)
        wait_fetch(1)
        compute(1)

    wait_fetch(0)
    compute(0)

    plsc.subcore_barrier()

    @pl.when(s < _NS - 1)
    def _():
        for b in range(_SPAN // _CK):
            r0 = s * _SPAN + b * _CK
            pltpu.sync_copy(u_sh.at[pl.ds(r0, _CK), :], exe_v)
            pltpu.sync_copy(exe_v, u_out.at[c, pl.ds(r0, _CK), :])

    @pl.when(s == _NS - 1)
    def _():
        for b in range(_SPAN_LAST // _CK):
            r0 = s * _SPAN + b * _CK
            pltpu.sync_copy(u_sh.at[pl.ds(r0, _CK), :], exe_v)
            pltpu.sync_copy(exe_v, u_out.at[c, pl.ds(r0, _CK), :])

    @pl.when(s < _NS - 1)
    def _():
        pltpu.sync_copy(sig_sh.at[pl.ds(s * _SPAN, _SPAN)], sigbuf_v)
        pltpu.sync_copy(sigbuf_v, sig_out.at[pl.ds(c * _N + s * _SPAN, _SPAN)])

    @pl.when(s == _NS - 1)
    def _():
        pltpu.sync_copy(sig_sh.at[pl.ds(s * _SPAN, _SPAN_LAST)],
                        sigbuf_v.at[pl.ds(0, _SPAN_LAST)])
        pltpu.sync_copy(sigbuf_v.at[pl.ds(0, _SPAN_LAST)],
                        sig_out.at[pl.ds(c * _N + s * _SPAN, _SPAN_LAST)])


_edge_kernel = pl.kernel(
    _edge_body,
    out_type=(jax.ShapeDtypeStruct((_NC, _N, _D), jnp.float32),
              jax.ShapeDtypeStruct((_NC * _N,), jnp.float32)),
    mesh=plsc.VectorSubcoreMesh(core_axis_name="c", subcore_axis_name="s",
                                num_cores=_NC, num_subcores=_NS),
    compiler_params=pltpu.CompilerParams(needs_layout_passes=False,
                                         use_tc_tiling_on_sc=False),
    scratch_types=(
        [pltpu.VMEM((_CK,), jnp.int32),
         pltpu.VMEM((_CK,), jnp.int32),
         pltpu.VMEM((_CK, _D // 2), jnp.int32),
         pltpu.VMEM((_CK, _D // 2), jnp.int32),
         pltpu.VMEM((_CK,), jnp.float32),
         pltpu.VMEM((_CK,), jnp.float32)]
        * 2
        + [pltpu.VMEM((_CK, _D), jnp.float32),
           pltpu.VMEM((_SPAN,), jnp.float32),
           pltpu.VMEM((_D, 16), jnp.float32),
           pltpu.VMEM_SHARED((_N, _D), jnp.float32),
           pltpu.VMEM_SHARED((_N,), jnp.float32)]
        + [pltpu.SemaphoreType.DMA] * 6
    ),
)


def _gru(x, h, Wz, Uz, Wr, Ur, Wn, Un):
    z = jax.nn.sigmoid(x @ Wz + h @ Uz)
    r = jax.nn.sigmoid(x @ Wr + h @ Ur)
    n = jnp.tanh(x @ Wn + r * (h @ Un))
    return (1.0 - z) * n + z * h


def _lrelu(x):
    return jnp.where(x >= 0, x, 0.01 * x)


def _elu(x):
    return jnp.where(x > 0, x, jnp.exp(jnp.minimum(x, 0.0)) - 1.0)


_EBLK = 8000


def _eb_body(fe_ref, we_ref, wo_ref, out_ref):
    fe = fe_ref[...]
    ye = jnp.dot(fe, we_ref[...], preferred_element_type=jnp.float32)
    yo = jnp.dot(fe, wo_ref[...], preferred_element_type=jnp.float32)
    ye_u = lax.bitcast_convert_type(ye.astype(jnp.bfloat16),
                                    jnp.uint16).astype(jnp.uint32)
    yo_u = lax.bitcast_convert_type(yo.astype(jnp.bfloat16),
                                    jnp.uint16).astype(jnp.uint32)
    out_ref[...] = lax.bitcast_convert_type(ye_u | (yo_u << 16), jnp.int32)


def _compute_eb(frag_edge, w_even, w_odd):
    return pl.pallas_call(
        _eb_body,
        grid=(_E // _EBLK,),
        in_specs=[pl.BlockSpec((_EBLK, _DE), lambda i: (i, 0)),
                  pl.BlockSpec((_DE, _D // 2), lambda i: (0, 0)),
                  pl.BlockSpec((_DE, _D // 2), lambda i: (0, 0))],
        out_specs=pl.BlockSpec((_EBLK, _D // 2), lambda i: (i, 0)),
        out_shape=jax.ShapeDtypeStruct((_E, _D // 2), jnp.int32),
    )(frag_edge, w_even, w_odd)


_NBLK = 1000


def _pack_pairs(he, ho):
    he_u = lax.bitcast_convert_type(he.astype(jnp.bfloat16),
                                    jnp.uint16).astype(jnp.uint32)
    ho_u = lax.bitcast_convert_type(ho.astype(jnp.bfloat16),
                                    jnp.uint16).astype(jnp.uint32)
    return lax.bitcast_convert_type(he_u | (ho_u << 16), jnp.int32)


def _init_body(x_ref, wi_ref, wee_ref, weo_ref, wa_ref, h_ref, hn_ref,
               hd_ref):
    h = _lrelu(jnp.dot(x_ref[...], wi_ref[...],
                       preferred_element_type=jnp.float32))
    h_ref[...] = h
    hn_ref[...] = _pack_pairs(
        jnp.dot(h, wee_ref[...], preferred_element_type=jnp.float32),
        jnp.dot(h, weo_ref[...], preferred_element_type=jnp.float32))
    hd_ref[...] = jnp.dot(h, wa_ref[...], preferred_element_type=jnp.float32)


def _compute_init(frag_node, W_init, We_even, We_odd, wa_top):
    return pl.pallas_call(
        _init_body,
        grid=(_N // _NBLK,),
        in_specs=[pl.BlockSpec((_NBLK, _D), lambda i: (i, 0)),
                  pl.BlockSpec((_D, _D), lambda i: (0, 0)),
                  pl.BlockSpec((_D, _D // 2), lambda i: (0, 0)),
                  pl.BlockSpec((_D, _D // 2), lambda i: (0, 0)),
                  pl.BlockSpec((_D, 1), lambda i: (0, 0))],
        out_specs=[pl.BlockSpec((_NBLK, _D), lambda i: (i, 0)),
                   pl.BlockSpec((_NBLK, _D // 2), lambda i: (i, 0)),
                   pl.BlockSpec((_NBLK, 1), lambda i: (i, 0))],
        out_shape=[jax.ShapeDtypeStruct((_N, _D), jnp.float32),
                   jax.ShapeDtypeStruct((_N, _D // 2), jnp.int32),
                   jax.ShapeDtypeStruct((_N, 1), jnp.float32)],
    )(frag_node, W_init, We_even, We_odd, wa_top)


def _update_body(u_ref, sig_ref, h_ref, wmsg_ref, wz_ref, uz_ref, wr_ref,
                 ur_ref, wn_ref, un_ref, wee_ref, weo_ref, wa_ref,
                 h_out, hn_out, hd_out):
    sig = sig_ref[:, 0] + sig_ref[:, 1]
    U = u_ref[0] + u_ref[1]
    S = U / (sig + 1e-9)[:, None]
    ctx = _elu(jnp.dot(S, wmsg_ref[...], preferred_element_type=jnp.float32))
    h = h_ref[...]
    z = jax.nn.sigmoid(jnp.dot(ctx, wz_ref[...], preferred_element_type=jnp.float32)
                       + jnp.dot(h, uz_ref[...], preferred_element_type=jnp.float32))
    r = jax.nn.sigmoid(jnp.dot(ctx, wr_ref[...], preferred_element_type=jnp.float32)
                       + jnp.dot(h, ur_ref[...], preferred_element_type=jnp.float32))
    n = jnp.tanh(jnp.dot(ctx, wn_ref[...], preferred_element_type=jnp.float32)
                 + r * jnp.dot(h, un_ref[...], preferred_element_type=jnp.float32))
    hn = (1.0 - z) * n + z * h
    h_out[...] = hn
    hn_out[...] = _pack_pairs(
        jnp.dot(hn, wee_ref[...], preferred_element_type=jnp.float32),
        jnp.dot(hn, weo_ref[...], preferred_element_type=jnp.float32))
    hd_out[...] = jnp.dot(hn, wa_ref[...], preferred_element_type=jnp.float32)


def _compute_update(U2, sig2, h, W_msg, Wz, Uz, Wr, Ur, Wn, Un, We_even,
                    We_odd, wa_top):
    wspec = pl.BlockSpec((_D, _D), lambda i: (0, 0))
    hspec = pl.BlockSpec((_D, _D // 2), lambda i: (0, 0))
    return pl.pallas_call(
        _update_body,
        grid=(_N // _NBLK,),
        in_specs=[pl.BlockSpec((_NC, _NBLK, _D), lambda i: (0, i, 0)),
                  pl.BlockSpec((_NBLK, _NC), lambda i: (i, 0)),
                  pl.BlockSpec((_NBLK, _D), lambda i: (i, 0)),
                  wspec, wspec, wspec, wspec, wspec, wspec, wspec,
                  hspec, hspec,
                  pl.BlockSpec((_D, 1), lambda i: (0, 0))],
        out_specs=[pl.BlockSpec((_NBLK, _D), lambda i: (i, 0)),
                   pl.BlockSpec((_NBLK, _D // 2), lambda i: (i, 0)),
                   pl.BlockSpec((_NBLK, 1), lambda i: (i, 0))],
        out_shape=[jax.ShapeDtypeStruct((_N, _D), jnp.float32),
                   jax.ShapeDtypeStruct((_N, _D // 2), jnp.int32),
                   jax.ShapeDtypeStruct((_N, 1), jnp.float32)],
    )(U2, sig2, h, W_msg, Wz, Uz, Wr, Ur, Wn, Un, We_even, We_odd, wa_top)


def _mol_body(h_ref, ids_ref, wmt_ref, wmb_ref, wmsg_ref, wz_ref, uz_ref,
              wr_ref, ur_ref, wn_ref, un_ref, g_out):
    h = h_ref[...]
    ids = ids_ref[...]
    iota_g = lax.broadcasted_iota(jnp.int32, (_G, _N), 0)
    M = (iota_g == ids).astype(jnp.float32)
    iota_n = lax.broadcasted_iota(jnp.int32, (_N, _G), 1)
    MT = (iota_n == ids.reshape(_N, 1)).astype(jnp.float32)
    g = jnp.dot(M, h, preferred_element_type=jnp.float32)
    wmb_row = wmb_ref[...]
    for _ in range(_T):
        gl = jnp.dot(g, wmt_ref[...], preferred_element_type=jnp.float32)
        hl = jnp.sum(h * wmb_row, axis=1, keepdims=True)
        glg = jnp.dot(MT, gl, preferred_element_type=jnp.float32)
        logit = _lrelu(glg + hl)
        ex = jnp.exp(logit)
        sig = jnp.dot(M, ex, preferred_element_type=jnp.float32)
        sigg = jnp.dot(MT, sig, preferred_element_type=jnp.float32)
        w = ex / (sigg + 1e-9)
        U = jnp.dot(M, w * h, preferred_element_type=jnp.float32)
        ctx = _elu(jnp.dot(U, wmsg_ref[...], preferred_element_type=jnp.float32))
        z = jax.nn.sigmoid(jnp.dot(ctx, wz_ref[...], preferred_element_type=jnp.float32)
                           + jnp.dot(g, uz_ref[...], preferred_element_type=jnp.float32))
        r = jax.nn.sigmoid(jnp.dot(ctx, wr_ref[...], preferred_element_type=jnp.float32)
                           + jnp.dot(g, ur_ref[...], preferred_element_type=jnp.float32))
        n = jnp.tanh(jnp.dot(ctx, wn_ref[...], preferred_element_type=jnp.float32)
                     + r * jnp.dot(g, un_ref[...], preferred_element_type=jnp.float32))
        g = (1.0 - z) * n + z * g
    g_out[...] = g


def _compute_mol(h, ids2d, wm_top, wm_bot_row, W_msg_m, Wz, Uz, Wr, Ur, Wn, Un):
    return pl.pallas_call(
        _mol_body,
        in_specs=[pl.BlockSpec((_N, _D), lambda: (0, 0)),
                  pl.BlockSpec((1, _N), lambda: (0, 0)),
                  pl.BlockSpec((_D, 1), lambda: (0, 0)),
                  pl.BlockSpec((1, _D), lambda: (0, 0)),
                  pl.BlockSpec((_D, _D), lambda: (0, 0)),
                  pl.BlockSpec((_D, _D), lambda: (0, 0)),
                  pl.BlockSpec((_D, _D), lambda: (0, 0)),
                  pl.BlockSpec((_D, _D), lambda: (0, 0)),
                  pl.BlockSpec((_D, _D), lambda: (0, 0)),
                  pl.BlockSpec((_D, _D), lambda: (0, 0)),
                  pl.BlockSpec((_D, _D), lambda: (0, 0))],
        out_specs=pl.BlockSpec((_G, _D), lambda: (0, 0)),
        out_shape=jax.ShapeDtypeStruct((_G, _D), jnp.float32),
    )(h, ids2d, wm_top, wm_bot_row, W_msg_m, Wz, Uz, Wr, Ur, Wn, Un)


def kernel(frag_node, frag_edge, edge_index, graph_ids, W_init, W_edge, w_att, W_msg,
           Wz_a, Uz_a, Wr_a, Ur_a, Wn_a, Un_a,
           w_att_m, W_msg_m, Wz_m, Uz_m, Wr_m, Ur_m, Wn_m, Un_m):
    src = edge_index[0]
    dst = edge_index[1]
    We_top = W_edge[:_D]
    wa_top = w_att[:_D]
    wa_bot = w_att[_D:, 0]
    W_msg_p = W_msg
    W_bot = W_edge[_D:]
    Ebi = _compute_eb(frag_edge, W_bot[:, 0::2], W_bot[:, 1::2])
    h, Hn, hd = _compute_init(frag_node, W_init, We_top[:, 0::2],
                              We_top[:, 1::2], wa_top)
    wa_bot_sp = jnp.broadcast_to(wa_bot[:, None], (_D, 16))
    for _ in range(_L):
        U2, sigf = _edge_kernel(src, dst, Hn, hd[:, 0], Ebi, wa_bot_sp)
        h, Hn, hd = _compute_update(U2, sigf.reshape(_NC, _N).T, h, W_msg_p,
                                    Wz_a, Uz_a, Wr_a, Ur_a, Wn_a, Un_a,
                                    We_top[:, 0::2], We_top[:, 1::2], wa_top)
    g = _compute_mol(h, graph_ids.reshape(1, _N), w_att_m[:_D],
                     w_att_m[_D:, 0].reshape(1, _D), W_msg_m,
                     Wz_m, Uz_m, Wr_m, Ur_m, Wn_m, Un_m)
    return g


# speedup vs baseline: 3.9728x; 3.9728x over previous
import jax
import jax.numpy as jnp
from jax import lax
from jax.experimental import pallas as pl
from jax.experimental.pallas import tpu as pltpu
from jax.experimental.pallas import tpu_sc as plsc

_N = 10000
_E = 320000
_D = 128
_DE = 16
_G = 256
_L = 2
_T = 2

_NC, _NS = 2, 16
_NW = _NC * _NS
_EW = _E // _NW
_CK = 80
_NCHUNK = _EW // _CK
_SPAN = 640
_SPAN_LAST = _N - (_NS - 1) * _SPAN
_NP = _NS * _SPAN


def _edge_body(src_h, dst_h, hn_h, hd_h, eb_h, w_h,
               u_out, sig_out,
               src_a, dst_a, rows_a, eb_a, hdb_a, exc_a,
               src_b, dst_b, rows_b, eb_b, hdb_b, exc_b,
               exe_v, sigbuf_v, w_l, u_sh, sig_sh,
               gsem_a, esem_a, hsem_a,
               gsem_b, esem_b, hsem_b):
    c = lax.axis_index("c")
    s = lax.axis_index("s")
    wid = c * _NS + s
    z16f = jnp.zeros((16,), jnp.float32)
    srcs = (src_a, src_b)
    dsts = (dst_a, dst_b)
    rows = (rows_a, rows_b)
    ebs = (eb_a, eb_b)
    hdbs = (hdb_a, hdb_b)
    excs = (exc_a, exc_b)
    gsems = (gsem_a, gsem_b)
    esems = (esem_a, esem_b)
    hsems = (hsem_a, hsem_b)

    @pl.loop(0, _CK)
    def _zrow(r):
        for k in range(8):
            exe_v[r, pl.ds(k * 16, 16)] = z16f
    for i in range(_SPAN // 16):
        sigbuf_v[pl.ds(i * 16, 16)] = z16f

    @pl.when(s < _NS - 1)
    def _():
        pltpu.sync_copy(sigbuf_v, sig_sh.at[pl.ds(s * _SPAN, _SPAN)])

    @pl.when(s == _NS - 1)
    def _():
        pltpu.sync_copy(sigbuf_v.at[pl.ds(0, _SPAN_LAST)],
                        sig_sh.at[pl.ds(s * _SPAN, _SPAN_LAST)])

    @pl.when(s < _NS - 1)
    def _():
        for b in range(_SPAN // _CK):
            pltpu.sync_copy(exe_v, u_sh.at[pl.ds(s * _SPAN + b * _CK, _CK), :])

    @pl.when(s == _NS - 1)
    def _():
        for b in range(_SPAN_LAST // _CK):
            pltpu.sync_copy(exe_v, u_sh.at[pl.ds(s * _SPAN + b * _CK, _CK), :])

    pltpu.sync_copy(w_h, w_l)
    plsc.subcore_barrier()

    iota16 = lax.iota(jnp.int32, 16)
    wvs = [w_l[pl.ds(k * 16, 16)] for k in range(8)]
    ebase = wid * _EW

    def fetch(bi, ci):
        base = ebase + ci * _CK
        pltpu.sync_copy(src_h.at[pl.ds(base, _CK)], srcs[bi])
        pltpu.sync_copy(dst_h.at[pl.ds(base, _CK)], dsts[bi])
        pltpu.async_copy(hn_h.at[srcs[bi]], rows[bi], gsems[bi])
        pltpu.async_copy(hd_h.at[dsts[bi]], hdbs[bi], hsems[bi])
        pltpu.async_copy(eb_h.at[pl.ds(base, _CK), :], ebs[bi], esems[bi])

    def wait_fetch(bi):
        pltpu.make_async_copy(hn_h.at[srcs[bi]], rows[bi], gsems[bi]).wait()
        pltpu.make_async_copy(hd_h.at[dsts[bi]], hdbs[bi], hsems[bi]).wait()
        pltpu.make_async_copy(eb_h.at[pl.ds(0, _CK), :], ebs[bi],
                              esems[bi]).wait()

    shuf_idx = [jnp.bitwise_xor(iota16, jnp.int32(off))
                for off in (8, 4, 2, 1)]

    def _allsum(v):
        for idx in shuf_idx:
            v = v + jnp.take_along_axis(v, idx, axis=0,
                                        mode="promise_in_bounds")
        return v

    def compute(bi):
        rows_v, eb_v = rows[bi], ebs[bi]
        dst_v, exc_v, hdb_v = dsts[bi], excs[bi], hdbs[bi]

        @pl.loop(0, _CK // 16)
        def _group(j):
            r0 = j * 16
            riv = r0 + iota16
            hdj = plsc.load_gather(hdb_v, [riv])
            sv = jnp.zeros((16,), jnp.float32)
            for e in range(16):
                row = r0 + e
                acc = None
                for kk in range(4):
                    ebw = eb_v[row, pl.ds(kk * 16, 16)]
                    ev0, ev1 = plsc.unpack(
                        plsc.bitcast(ebw, jnp.bfloat16),
                        format=plsc.PackFormat.INTERLEAVED)
                    hnw = rows_v[row, pl.ds(kk * 16, 16)]
                    hv0, hv1 = plsc.unpack(
                        plsc.bitcast(hnw, jnp.bfloat16),
                        format=plsc.PackFormat.INTERLEAVED)
                    for half, (hv, ev) in enumerate(((hv0, ev0), (hv1, ev1))):
                        k = 2 * kk + half
                        x = hv + ev
                        evec = jnp.where(x >= 0, x, 0.01 * x)
                        exe_v[row, pl.ds(k * 16, 16)] = evec
                        acc = (evec * wvs[k] if acc is None
                               else acc + evec * wvs[k])
                sv = jnp.where(iota16 == e, _allsum(acc), sv)
            lg = hdj + sv
            logit = jnp.where(lg >= 0, lg, 0.01 * lg)
            ex = jnp.exp(logit)
            plsc.store_scatter(exc_v, [riv], ex)
            for e in range(16):
                row = r0 + e
                exs = jnp.take_along_axis(ex, jnp.full((16,), e, jnp.int32),
                                          axis=0, mode="promise_in_bounds")
                for k in range(8):
                    exe_v[row, pl.ds(k * 16, 16)] = (
                        exe_v[row, pl.ds(k * 16, 16)] * exs)

        pltpu.sync_copy(exe_v, u_sh.at[dst_v], add=True)
        pltpu.sync_copy(exc_v, sig_sh.at[dst_v], add=True)

    fetch(0, 0)

    @pl.loop(0, (_NCHUNK - 1) // 2)
    def _pair(p):
        c0 = 2 * p
        fetch(1, c0 + 1)
        wait_fetch(0)
        compute(0)
        fetch(0, c0 + 2)
        wait_fetch(1)
        compute(1)

    wait_fetch(0)
    compute(0)

    plsc.subcore_barrier()

    @pl.when(s < _NS - 1)
    def _():
        for b in range(_SPAN // _CK):
            r0 = s * _SPAN + b * _CK
            pltpu.sync_copy(u_sh.at[pl.ds(r0, _CK), :], exe_v)
            pltpu.sync_copy(exe_v, u_out.at[c, pl.ds(r0, _CK), :])

    @pl.when(s == _NS - 1)
    def _():
        for b in range(_SPAN_LAST // _CK):
            r0 = s * _SPAN + b * _CK
            pltpu.sync_copy(u_sh.at[pl.ds(r0, _CK), :], exe_v)
            pltpu.sync_copy(exe_v, u_out.at[c, pl.ds(r0, _CK), :])

    @pl.when(s < _NS - 1)
    def _():
        pltpu.sync_copy(sig_sh.at[pl.ds(s * _SPAN, _SPAN)], sigbuf_v)
        pltpu.sync_copy(sigbuf_v, sig_out.at[pl.ds(c * _N + s * _SPAN, _SPAN)])

    @pl.when(s == _NS - 1)
    def _():
        pltpu.sync_copy(sig_sh.at[pl.ds(s * _SPAN, _SPAN_LAST)],
                        sigbuf_v.at[pl.ds(0, _SPAN_LAST)])
        pltpu.sync_copy(sigbuf_v.at[pl.ds(0, _SPAN_LAST)],
                        sig_out.at[pl.ds(c * _N + s * _SPAN, _SPAN_LAST)])


_edge_kernel = pl.kernel(
    _edge_body,
    out_type=(jax.ShapeDtypeStruct((_NC, _N, _D), jnp.float32),
              jax.ShapeDtypeStruct((_NC * _N,), jnp.float32)),
    mesh=plsc.VectorSubcoreMesh(core_axis_name="c", subcore_axis_name="s",
                                num_cores=_NC, num_subcores=_NS),
    compiler_params=pltpu.CompilerParams(needs_layout_passes=False,
                                         use_tc_tiling_on_sc=False),
    scratch_types=(
        [pltpu.VMEM((_CK,), jnp.int32),
         pltpu.VMEM((_CK,), jnp.int32),
         pltpu.VMEM((_CK, _D // 2), jnp.int32),
         pltpu.VMEM((_CK, _D // 2), jnp.int32),
         pltpu.VMEM((_CK,), jnp.float32),
         pltpu.VMEM((_CK,), jnp.float32)]
        * 2
        + [pltpu.VMEM((_CK, _D), jnp.float32),
           pltpu.VMEM((_SPAN,), jnp.float32),
           pltpu.VMEM((_D,), jnp.float32),
           pltpu.VMEM_SHARED((_N, _D), jnp.float32),
           pltpu.VMEM_SHARED((_N,), jnp.float32)]
        + [pltpu.SemaphoreType.DMA] * 6
    ),
)


def _gru(x, h, Wz, Uz, Wr, Ur, Wn, Un):
    z = jax.nn.sigmoid(x @ Wz + h @ Uz)
    r = jax.nn.sigmoid(x @ Wr + h @ Ur)
    n = jnp.tanh(x @ Wn + r * (h @ Un))
    return (1.0 - z) * n + z * h


def _lrelu(x):
    return jnp.where(x >= 0, x, 0.01 * x)


def _elu(x):
    return jnp.where(x > 0, x, jnp.exp(jnp.minimum(x, 0.0)) - 1.0)


_EBLK = 8000


def _eb_body(fe_ref, we_ref, wo_ref, out_ref):
    fe = fe_ref[...]
    ye = jnp.dot(fe, we_ref[...], preferred_element_type=jnp.float32)
    yo = jnp.dot(fe, wo_ref[...], preferred_element_type=jnp.float32)
    ye_u = lax.bitcast_convert_type(ye.astype(jnp.bfloat16),
                                    jnp.uint16).astype(jnp.uint32)
    yo_u = lax.bitcast_convert_type(yo.astype(jnp.bfloat16),
                                    jnp.uint16).astype(jnp.uint32)
    out_ref[...] = lax.bitcast_convert_type(ye_u | (yo_u << 16), jnp.int32)


def _compute_eb(frag_edge, w_even, w_odd):
    return pl.pallas_call(
        _eb_body,
        grid=(_E // _EBLK,),
        in_specs=[pl.BlockSpec((_EBLK, _DE), lambda i: (i, 0)),
                  pl.BlockSpec((_DE, _D // 2), lambda i: (0, 0)),
                  pl.BlockSpec((_DE, _D // 2), lambda i: (0, 0))],
        out_specs=pl.BlockSpec((_EBLK, _D // 2), lambda i: (i, 0)),
        out_shape=jax.ShapeDtypeStruct((_E, _D // 2), jnp.int32),
    )(frag_edge, w_even, w_odd)


_NBLK = 1000


def _pack_pairs(he, ho):
    he_u = lax.bitcast_convert_type(he.astype(jnp.bfloat16),
                                    jnp.uint16).astype(jnp.uint32)
    ho_u = lax.bitcast_convert_type(ho.astype(jnp.bfloat16),
                                    jnp.uint16).astype(jnp.uint32)
    return lax.bitcast_convert_type(he_u | (ho_u << 16), jnp.int32)


def _init_body(x_ref, wi_ref, wee_ref, weo_ref, wa_ref, h_ref, hn_ref,
               hd_ref):
    h = _lrelu(jnp.dot(x_ref[...], wi_ref[...],
                       preferred_element_type=jnp.float32))
    h_ref[...] = h
    hn_ref[...] = _pack_pairs(
        jnp.dot(h, wee_ref[...], preferred_element_type=jnp.float32),
        jnp.dot(h, weo_ref[...], preferred_element_type=jnp.float32))
    hd_ref[...] = jnp.dot(h, wa_ref[...], preferred_element_type=jnp.float32)


def _compute_init(frag_node, W_init, We_even, We_odd, wa_top):
    return pl.pallas_call(
        _init_body,
        grid=(_N // _NBLK,),
        in_specs=[pl.BlockSpec((_NBLK, _D), lambda i: (i, 0)),
                  pl.BlockSpec((_D, _D), lambda i: (0, 0)),
                  pl.BlockSpec((_D, _D // 2), lambda i: (0, 0)),
                  pl.BlockSpec((_D, _D // 2), lambda i: (0, 0)),
                  pl.BlockSpec((_D, 1), lambda i: (0, 0))],
        out_specs=[pl.BlockSpec((_NBLK, _D), lambda i: (i, 0)),
                   pl.BlockSpec((_NBLK, _D // 2), lambda i: (i, 0)),
                   pl.BlockSpec((_NBLK, 1), lambda i: (i, 0))],
        out_shape=[jax.ShapeDtypeStruct((_N, _D), jnp.float32),
                   jax.ShapeDtypeStruct((_N, _D // 2), jnp.int32),
                   jax.ShapeDtypeStruct((_N, 1), jnp.float32)],
    )(frag_node, W_init, We_even, We_odd, wa_top)


def _update_body(u_ref, sig_ref, h_ref, wmsg_ref, wz_ref, uz_ref, wr_ref,
                 ur_ref, wn_ref, un_ref, wee_ref, weo_ref, wa_ref,
                 h_out, hn_out, hd_out):
    sig = sig_ref[:, 0] + sig_ref[:, 1]
    U = u_ref[0] + u_ref[1]
    S = U / (sig + 1e-9)[:, None]
    ctx = _elu(jnp.dot(S, wmsg_ref[...], preferred_element_type=jnp.float32))
    h = h_ref[...]
    z = jax.nn.sigmoid(jnp.dot(ctx, wz_ref[...], preferred_element_type=jnp.float32)
                       + jnp.dot(h, uz_ref[...], preferred_element_type=jnp.float32))
    r = jax.nn.sigmoid(jnp.dot(ctx, wr_ref[...], preferred_element_type=jnp.float32)
                       + jnp.dot(h, ur_ref[...], preferred_element_type=jnp.float32))
    n = jnp.tanh(jnp.dot(ctx, wn_ref[...], preferred_element_type=jnp.float32)
                 + r * jnp.dot(h, un_ref[...], preferred_element_type=jnp.float32))
    hn = (1.0 - z) * n + z * h
    h_out[...] = hn
    hn_out[...] = _pack_pairs(
        jnp.dot(hn, wee_ref[...], preferred_element_type=jnp.float32),
        jnp.dot(hn, weo_ref[...], preferred_element_type=jnp.float32))
    hd_out[...] = jnp.dot(hn, wa_ref[...], preferred_element_type=jnp.float32)


def _compute_update(U2, sig2, h, W_msg, Wz, Uz, Wr, Ur, Wn, Un, We_even,
                    We_odd, wa_top):
    wspec = pl.BlockSpec((_D, _D), lambda i: (0, 0))
    hspec = pl.BlockSpec((_D, _D // 2), lambda i: (0, 0))
    return pl.pallas_call(
        _update_body,
        grid=(_N // _NBLK,),
        in_specs=[pl.BlockSpec((_NC, _NBLK, _D), lambda i: (0, i, 0)),
                  pl.BlockSpec((_NBLK, _NC), lambda i: (i, 0)),
                  pl.BlockSpec((_NBLK, _D), lambda i: (i, 0)),
                  wspec, wspec, wspec, wspec, wspec, wspec, wspec,
                  hspec, hspec,
                  pl.BlockSpec((_D, 1), lambda i: (0, 0))],
        out_specs=[pl.BlockSpec((_NBLK, _D), lambda i: (i, 0)),
                   pl.BlockSpec((_NBLK, _D // 2), lambda i: (i, 0)),
                   pl.BlockSpec((_NBLK, 1), lambda i: (i, 0))],
        out_shape=[jax.ShapeDtypeStruct((_N, _D), jnp.float32),
                   jax.ShapeDtypeStruct((_N, _D // 2), jnp.int32),
                   jax.ShapeDtypeStruct((_N, 1), jnp.float32)],
    )(U2, sig2, h, W_msg, Wz, Uz, Wr, Ur, Wn, Un, We_even, We_odd, wa_top)


def _mol_body(h_ref, ids_ref, wmt_ref, wmb_ref, wmsg_ref, wz_ref, uz_ref,
              wr_ref, ur_ref, wn_ref, un_ref, g_out):
    h = h_ref[...]
    ids = ids_ref[...]
    iota_g = lax.broadcasted_iota(jnp.int32, (_G, _N), 0)
    M = (iota_g == ids).astype(jnp.float32)
    iota_n = lax.broadcasted_iota(jnp.int32, (_N, _G), 1)
    MT = (iota_n == ids.reshape(_N, 1)).astype(jnp.float32)
    g = jnp.dot(M, h, preferred_element_type=jnp.float32)
    wmb_row = wmb_ref[...]
    for _ in range(_T):
        gl = jnp.dot(g, wmt_ref[...], preferred_element_type=jnp.float32)
        hl = jnp.sum(h * wmb_row, axis=1, keepdims=True)
        glg = jnp.dot(MT, gl, preferred_element_type=jnp.float32)
        logit = _lrelu(glg + hl)
        ex = jnp.exp(logit)
        sig = jnp.dot(M, ex, preferred_element_type=jnp.float32)
        sigg = jnp.dot(MT, sig, preferred_element_type=jnp.float32)
        w = ex / (sigg + 1e-9)
        U = jnp.dot(M, w * h, preferred_element_type=jnp.float32)
        ctx = _elu(jnp.dot(U, wmsg_ref[...], preferred_element_type=jnp.float32))
        z = jax.nn.sigmoid(jnp.dot(ctx, wz_ref[...], preferred_element_type=jnp.float32)
                           + jnp.dot(g, uz_ref[...], preferred_element_type=jnp.float32))
        r = jax.nn.sigmoid(jnp.dot(ctx, wr_ref[...], preferred_element_type=jnp.float32)
                           + jnp.dot(g, ur_ref[...], preferred_element_type=jnp.float32))
        n = jnp.tanh(jnp.dot(ctx, wn_ref[...], preferred_element_type=jnp.float32)
                     + r * jnp.dot(g, un_ref[...], preferred_element_type=jnp.float32))
        g = (1.0 - z) * n + z * g
    g_out[...] = g


def _compute_mol(h, ids2d, wm_top, wm_bot_row, W_msg_m, Wz, Uz, Wr, Ur, Wn, Un):
    return pl.pallas_call(
        _mol_body,
        in_specs=[pl.BlockSpec((_N, _D), lambda: (0, 0)),
                  pl.BlockSpec((1, _N), lambda: (0, 0)),
                  pl.BlockSpec((_D, 1), lambda: (0, 0)),
                  pl.BlockSpec((1, _D), lambda: (0, 0)),
                  pl.BlockSpec((_D, _D), lambda: (0, 0)),
                  pl.BlockSpec((_D, _D), lambda: (0, 0)),
                  pl.BlockSpec((_D, _D), lambda: (0, 0)),
                  pl.BlockSpec((_D, _D), lambda: (0, 0)),
                  pl.BlockSpec((_D, _D), lambda: (0, 0)),
                  pl.BlockSpec((_D, _D), lambda: (0, 0)),
                  pl.BlockSpec((_D, _D), lambda: (0, 0))],
        out_specs=pl.BlockSpec((_G, _D), lambda: (0, 0)),
        out_shape=jax.ShapeDtypeStruct((_G, _D), jnp.float32),
    )(h, ids2d, wm_top, wm_bot_row, W_msg_m, Wz, Uz, Wr, Ur, Wn, Un)


def kernel(frag_node, frag_edge, edge_index, graph_ids, W_init, W_edge, w_att, W_msg,
           Wz_a, Uz_a, Wr_a, Ur_a, Wn_a, Un_a,
           w_att_m, W_msg_m, Wz_m, Uz_m, Wr_m, Ur_m, Wn_m, Un_m):
    src = edge_index[0]
    dst = edge_index[1]
    perm = []
    for kk in range(_D // 32):
        perm += list(range(32 * kk, 32 * kk + 32, 2))
        perm += list(range(32 * kk + 1, 32 * kk + 32, 2))
    perm = jnp.array(perm, jnp.int32)
    We_top = W_edge[:_D]
    wa_top = w_att[:_D]
    wa_bot = w_att[_D:, 0][perm]
    W_msg_p = W_msg[perm]
    W_bot = W_edge[_D:]
    Ebi = _compute_eb(frag_edge, W_bot[:, 0::2], W_bot[:, 1::2])
    h, Hn, hd = _compute_init(frag_node, W_init, We_top[:, 0::2],
                              We_top[:, 1::2], wa_top)
    for _ in range(_L):
        U2, sigf = _edge_kernel(src, dst, Hn, hd[:, 0], Ebi, wa_bot)
        h, Hn, hd = _compute_update(U2, sigf.reshape(_NC, _N).T, h, W_msg_p,
                                    Wz_a, Uz_a, Wr_a, Ur_a, Wn_a, Un_a,
                                    We_top[:, 0::2], We_top[:, 1::2], wa_top)
    g = _compute_mol(h, graph_ids.reshape(1, _N), w_att_m[:_D],
                     w_att_m[_D:, 0].reshape(1, _D), W_msg_m,
                     Wz_m, Uz_m, Wr_m, Ur_m, Wn_m, Un_m)
    return g


# speedup vs baseline: 5.8685x; 1.4772x over previous
import jax
import jax.numpy as jnp
from jax import lax
from jax.experimental import pallas as pl
from jax.experimental.pallas import tpu as pltpu
from jax.experimental.pallas import tpu_sc as plsc

_N = 10000
_E = 320000
_D = 128
_DE = 16
_G = 256
_L = 2
_T = 2

_NC, _NS = 2, 16
_NW = _NC * _NS
_EW = _E // _NW
_CK = 80
_NCHUNK = _EW // _CK
_SPAN = 640
_SPAN_LAST = _N - (_NS - 1) * _SPAN
_NP = _NS * _SPAN


def _edge_body(src_h, dst_h, hn_h, hd_h, eb_h, w_h,
               u_out, sig_out,
               src_a, dst_a, rows_a, eb_a, hdb_a, exc_a,
               src_b, dst_b, rows_b, eb_b, hdb_b, exc_b,
               exe_v, sigbuf_v, w_l, u_sh, sig_sh,
               gsem_a, esem_a, hsem_a,
               gsem_b, esem_b, hsem_b):
    c = lax.axis_index("c")
    s = lax.axis_index("s")
    wid = c * _NS + s
    z16f = jnp.zeros((16,), jnp.float32)
    srcs = (src_a, src_b)
    dsts = (dst_a, dst_b)
    rows = (rows_a, rows_b)
    ebs = (eb_a, eb_b)
    hdbs = (hdb_a, hdb_b)
    excs = (exc_a, exc_b)
    gsems = (gsem_a, gsem_b)
    esems = (esem_a, esem_b)
    hsems = (hsem_a, hsem_b)

    @pl.loop(0, _CK)
    def _zrow(r):
        for k in range(8):
            exe_v[r, pl.ds(k * 16, 16)] = z16f
    for i in range(_SPAN // 16):
        sigbuf_v[pl.ds(i * 16, 16)] = z16f

    @pl.when(s < _NS - 1)
    def _():
        pltpu.sync_copy(sigbuf_v, sig_sh.at[pl.ds(s * _SPAN, _SPAN)])

    @pl.when(s == _NS - 1)
    def _():
        pltpu.sync_copy(sigbuf_v.at[pl.ds(0, _SPAN_LAST)],
                        sig_sh.at[pl.ds(s * _SPAN, _SPAN_LAST)])

    @pl.when(s < _NS - 1)
    def _():
        for b in range(_SPAN // _CK):
            pltpu.sync_copy(exe_v, u_sh.at[pl.ds(s * _SPAN + b * _CK, _CK), :])

    @pl.when(s == _NS - 1)
    def _():
        for b in range(_SPAN_LAST // _CK):
            pltpu.sync_copy(exe_v, u_sh.at[pl.ds(s * _SPAN + b * _CK, _CK), :])

    pltpu.sync_copy(w_h, w_l)
    plsc.subcore_barrier()

    iota16 = lax.iota(jnp.int32, 16)
    wvs = [w_l[pl.ds(k * 16, 16)] for k in range(8)]
    ebase = wid * _EW

    def fetch(bi, ci):
        base = ebase + ci * _CK
        pltpu.sync_copy(src_h.at[pl.ds(base, _CK)], srcs[bi])
        pltpu.sync_copy(dst_h.at[pl.ds(base, _CK)], dsts[bi])
        pltpu.async_copy(hn_h.at[srcs[bi]], rows[bi], gsems[bi])
        pltpu.async_copy(hd_h.at[dsts[bi]], hdbs[bi], hsems[bi])
        pltpu.async_copy(eb_h.at[pl.ds(base, _CK), :], ebs[bi], esems[bi])

    def wait_fetch(bi):
        pltpu.make_async_copy(hn_h.at[srcs[bi]], rows[bi], gsems[bi]).wait()
        pltpu.make_async_copy(hd_h.at[dsts[bi]], hdbs[bi], hsems[bi]).wait()
        pltpu.make_async_copy(eb_h.at[pl.ds(0, _CK), :], ebs[bi],
                              esems[bi]).wait()

    shuf_idx = [jnp.bitwise_xor(iota16, jnp.int32(off))
                for off in (8, 4, 2, 1)]

    def _allsum(v):
        for idx in shuf_idx:
            v = v + jnp.take_along_axis(v, idx, axis=0,
                                        mode="promise_in_bounds")
        return v

    mask0 = iota16 == 0

    def compute(bi):
        rows_v, eb_v = rows[bi], ebs[bi]
        dst_v, exc_v, hdb_v = dsts[bi], excs[bi], hdbs[bi]

        @plsc.parallel_loop(0, _CK, unroll=4)
        def _edge(r):
            rsp = jnp.full((16,), r, jnp.int32)
            hd_e = plsc.load_gather(hdb_v, [rsp])
            acc = None
            evecs = []
            for kk in range(4):
                ebw = eb_v[r, pl.ds(kk * 16, 16)]
                ev0, ev1 = plsc.unpack(plsc.bitcast(ebw, jnp.bfloat16),
                                       format=plsc.PackFormat.INTERLEAVED)
                hnw = rows_v[r, pl.ds(kk * 16, 16)]
                hv0, hv1 = plsc.unpack(plsc.bitcast(hnw, jnp.bfloat16),
                                       format=plsc.PackFormat.INTERLEAVED)
                for half, (hv, ev) in enumerate(((hv0, ev0), (hv1, ev1))):
                    k = 2 * kk + half
                    x = hv + ev
                    evec = jnp.where(x >= 0, x, 0.01 * x)
                    evecs.append(evec)
                    acc = (evec * wvs[k] if acc is None
                           else acc + evec * wvs[k])
            s = _allsum(acc)
            lg = hd_e + s
            logit = jnp.where(lg >= 0, lg, 0.01 * lg)
            ex = jnp.exp(logit)
            plsc.store_scatter(exc_v, [rsp], ex, mask=mask0)
            for k in range(8):
                exe_v[r, pl.ds(k * 16, 16)] = evecs[k] * ex

        pltpu.sync_copy(exe_v, u_sh.at[dst_v], add=True)
        pltpu.sync_copy(exc_v, sig_sh.at[dst_v], add=True)

    fetch(0, 0)

    @pl.loop(0, (_NCHUNK - 1) // 2)
    def _pair(p):
        c0 = 2 * p
        fetch(1, c0 + 1)
        wait_fetch(0)
        compute(0)
        fetch(0, c0 + 2)
        wait_fetch(1)
        compute(1)

    wait_fetch(0)
    compute(0)

    plsc.subcore_barrier()

    @pl.when(s < _NS - 1)
    def _():
        for b in range(_SPAN // _CK):
            r0 = s * _SPAN + b * _CK
            pltpu.sync_copy(u_sh.at[pl.ds(r0, _CK), :], exe_v)
            pltpu.sync_copy(exe_v, u_out.at[c, pl.ds(r0, _CK), :])

    @pl.when(s == _NS - 1)
    def _():
        for b in range(_SPAN_LAST // _CK):
            r0 = s * _SPAN + b * _CK
            pltpu.sync_copy(u_sh.at[pl.ds(r0, _CK), :], exe_v)
            pltpu.sync_copy(exe_v, u_out.at[c, pl.ds(r0, _CK), :])

    @pl.when(s < _NS - 1)
    def _():
        pltpu.sync_copy(sig_sh.at[pl.ds(s * _SPAN, _SPAN)], sigbuf_v)
        pltpu.sync_copy(sigbuf_v, sig_out.at[pl.ds(c * _N + s * _SPAN, _SPAN)])

    @pl.when(s == _NS - 1)
    def _():
        pltpu.sync_copy(sig_sh.at[pl.ds(s * _SPAN, _SPAN_LAST)],
                        sigbuf_v.at[pl.ds(0, _SPAN_LAST)])
        pltpu.sync_copy(sigbuf_v.at[pl.ds(0, _SPAN_LAST)],
                        sig_out.at[pl.ds(c * _N + s * _SPAN, _SPAN_LAST)])


_edge_kernel = pl.kernel(
    _edge_body,
    out_type=(jax.ShapeDtypeStruct((_NC, _N, _D), jnp.float32),
              jax.ShapeDtypeStruct((_NC * _N,), jnp.float32)),
    mesh=plsc.VectorSubcoreMesh(core_axis_name="c", subcore_axis_name="s",
                                num_cores=_NC, num_subcores=_NS),
    compiler_params=pltpu.CompilerParams(needs_layout_passes=False,
                                         use_tc_tiling_on_sc=False),
    scratch_types=(
        [pltpu.VMEM((_CK,), jnp.int32),
         pltpu.VMEM((_CK,), jnp.int32),
         pltpu.VMEM((_CK, _D // 2), jnp.int32),
         pltpu.VMEM((_CK, _D // 2), jnp.int32),
         pltpu.VMEM((_CK,), jnp.float32),
         pltpu.VMEM((_CK,), jnp.float32)]
        * 2
        + [pltpu.VMEM((_CK, _D), jnp.float32),
           pltpu.VMEM((_SPAN,), jnp.float32),
           pltpu.VMEM((_D,), jnp.float32),
           pltpu.VMEM_SHARED((_N, _D), jnp.float32),
           pltpu.VMEM_SHARED((_N,), jnp.float32)]
        + [pltpu.SemaphoreType.DMA] * 6
    ),
)


def _gru(x, h, Wz, Uz, Wr, Ur, Wn, Un):
    z = jax.nn.sigmoid(x @ Wz + h @ Uz)
    r = jax.nn.sigmoid(x @ Wr + h @ Ur)
    n = jnp.tanh(x @ Wn + r * (h @ Un))
    return (1.0 - z) * n + z * h


def _lrelu(x):
    return jnp.where(x >= 0, x, 0.01 * x)


def _elu(x):
    return jnp.where(x > 0, x, jnp.exp(jnp.minimum(x, 0.0)) - 1.0)


_EBLK = 8000


def _eb_body(fe_ref, we_ref, wo_ref, out_ref):
    fe = fe_ref[...]
    ye = jnp.dot(fe, we_ref[...], preferred_element_type=jnp.float32)
    yo = jnp.dot(fe, wo_ref[...], preferred_element_type=jnp.float32)
    ye_u = lax.bitcast_convert_type(ye.astype(jnp.bfloat16),
                                    jnp.uint16).astype(jnp.uint32)
    yo_u = lax.bitcast_convert_type(yo.astype(jnp.bfloat16),
                                    jnp.uint16).astype(jnp.uint32)
    out_ref[...] = lax.bitcast_convert_type(ye_u | (yo_u << 16), jnp.int32)


def _compute_eb(frag_edge, w_even, w_odd):
    return pl.pallas_call(
        _eb_body,
        grid=(_E // _EBLK,),
        in_specs=[pl.BlockSpec((_EBLK, _DE), lambda i: (i, 0)),
                  pl.BlockSpec((_DE, _D // 2), lambda i: (0, 0)),
                  pl.BlockSpec((_DE, _D // 2), lambda i: (0, 0))],
        out_specs=pl.BlockSpec((_EBLK, _D // 2), lambda i: (i, 0)),
        out_shape=jax.ShapeDtypeStruct((_E, _D // 2), jnp.int32),
    )(frag_edge, w_even, w_odd)


_NBLK = 1000


def _pack_pairs(he, ho):
    he_u = lax.bitcast_convert_type(he.astype(jnp.bfloat16),
                                    jnp.uint16).astype(jnp.uint32)
    ho_u = lax.bitcast_convert_type(ho.astype(jnp.bfloat16),
                                    jnp.uint16).astype(jnp.uint32)
    return lax.bitcast_convert_type(he_u | (ho_u << 16), jnp.int32)


def _init_body(x_ref, wi_ref, wee_ref, weo_ref, wa_ref, h_ref, hn_ref,
               hd_ref):
    h = _lrelu(jnp.dot(x_ref[...], wi_ref[...],
                       preferred_element_type=jnp.float32))
    h_ref[...] = h
    hn_ref[...] = _pack_pairs(
        jnp.dot(h, wee_ref[...], preferred_element_type=jnp.float32),
        jnp.dot(h, weo_ref[...], preferred_element_type=jnp.float32))
    hd_ref[...] = jnp.dot(h, wa_ref[...], preferred_element_type=jnp.float32)


def _compute_init(frag_node, W_init, We_even, We_odd, wa_top):
    return pl.pallas_call(
        _init_body,
        grid=(_N // _NBLK,),
        in_specs=[pl.BlockSpec((_NBLK, _D), lambda i: (i, 0)),
                  pl.BlockSpec((_D, _D), lambda i: (0, 0)),
                  pl.BlockSpec((_D, _D // 2), lambda i: (0, 0)),
                  pl.BlockSpec((_D, _D // 2), lambda i: (0, 0)),
                  pl.BlockSpec((_D, 1), lambda i: (0, 0))],
        out_specs=[pl.BlockSpec((_NBLK, _D), lambda i: (i, 0)),
                   pl.BlockSpec((_NBLK, _D // 2), lambda i: (i, 0)),
                   pl.BlockSpec((_NBLK, 1), lambda i: (i, 0))],
        out_shape=[jax.ShapeDtypeStruct((_N, _D), jnp.float32),
                   jax.ShapeDtypeStruct((_N, _D // 2), jnp.int32),
                   jax.ShapeDtypeStruct((_N, 1), jnp.float32)],
    )(frag_node, W_init, We_even, We_odd, wa_top)


def _update_body(u_ref, sig_ref, h_ref, wmsg_ref, wz_ref, uz_ref, wr_ref,
                 ur_ref, wn_ref, un_ref, wee_ref, weo_ref, wa_ref,
                 h_out, hn_out, hd_out):
    sig = sig_ref[:, 0] + sig_ref[:, 1]
    U = u_ref[0] + u_ref[1]
    S = U / (sig + 1e-9)[:, None]
    ctx = _elu(jnp.dot(S, wmsg_ref[...], preferred_element_type=jnp.float32))
    h = h_ref[...]
    z = jax.nn.sigmoid(jnp.dot(ctx, wz_ref[...], preferred_element_type=jnp.float32)
                       + jnp.dot(h, uz_ref[...], preferred_element_type=jnp.float32))
    r = jax.nn.sigmoid(jnp.dot(ctx, wr_ref[...], preferred_element_type=jnp.float32)
                       + jnp.dot(h, ur_ref[...], preferred_element_type=jnp.float32))
    n = jnp.tanh(jnp.dot(ctx, wn_ref[...], preferred_element_type=jnp.float32)
                 + r * jnp.dot(h, un_ref[...], preferred_element_type=jnp.float32))
    hn = (1.0 - z) * n + z * h
    h_out[...] = hn
    hn_out[...] = _pack_pairs(
        jnp.dot(hn, wee_ref[...], preferred_element_type=jnp.float32),
        jnp.dot(hn, weo_ref[...], preferred_element_type=jnp.float32))
    hd_out[...] = jnp.dot(hn, wa_ref[...], preferred_element_type=jnp.float32)


def _compute_update(U2, sig2, h, W_msg, Wz, Uz, Wr, Ur, Wn, Un, We_even,
                    We_odd, wa_top):
    wspec = pl.BlockSpec((_D, _D), lambda i: (0, 0))
    hspec = pl.BlockSpec((_D, _D // 2), lambda i: (0, 0))
    return pl.pallas_call(
        _update_body,
        grid=(_N // _NBLK,),
        in_specs=[pl.BlockSpec((_NC, _NBLK, _D), lambda i: (0, i, 0)),
                  pl.BlockSpec((_NBLK, _NC), lambda i: (i, 0)),
                  pl.BlockSpec((_NBLK, _D), lambda i: (i, 0)),
                  wspec, wspec, wspec, wspec, wspec, wspec, wspec,
                  hspec, hspec,
                  pl.BlockSpec((_D, 1), lambda i: (0, 0))],
        out_specs=[pl.BlockSpec((_NBLK, _D), lambda i: (i, 0)),
                   pl.BlockSpec((_NBLK, _D // 2), lambda i: (i, 0)),
                   pl.BlockSpec((_NBLK, 1), lambda i: (i, 0))],
        out_shape=[jax.ShapeDtypeStruct((_N, _D), jnp.float32),
                   jax.ShapeDtypeStruct((_N, _D // 2), jnp.int32),
                   jax.ShapeDtypeStruct((_N, 1), jnp.float32)],
    )(U2, sig2, h, W_msg, Wz, Uz, Wr, Ur, Wn, Un, We_even, We_odd, wa_top)


def _mol_body(h_ref, ids_ref, wmt_ref, wmb_ref, wmsg_ref, wz_ref, uz_ref,
              wr_ref, ur_ref, wn_ref, un_ref, g_out):
    h = h_ref[...]
    ids = ids_ref[...]
    iota_g = lax.broadcasted_iota(jnp.int32, (_G, _N), 0)
    M = (iota_g == ids).astype(jnp.float32)
    iota_n = lax.broadcasted_iota(jnp.int32, (_N, _G), 1)
    MT = (iota_n == ids.reshape(_N, 1)).astype(jnp.float32)
    g = jnp.dot(M, h, preferred_element_type=jnp.float32)
    wmb_row = wmb_ref[...]
    for _ in range(_T):
        gl = jnp.dot(g, wmt_ref[...], preferred_element_type=jnp.float32)
        hl = jnp.sum(h * wmb_row, axis=1, keepdims=True)
        glg = jnp.dot(MT, gl, preferred_element_type=jnp.float32)
        logit = _lrelu(glg + hl)
        ex = jnp.exp(logit)
        sig = jnp.dot(M, ex, preferred_element_type=jnp.float32)
        sigg = jnp.dot(MT, sig, preferred_element_type=jnp.float32)
        w = ex / (sigg + 1e-9)
        U = jnp.dot(M, w * h, preferred_element_type=jnp.float32)
        ctx = _elu(jnp.dot(U, wmsg_ref[...], preferred_element_type=jnp.float32))
        z = jax.nn.sigmoid(jnp.dot(ctx, wz_ref[...], preferred_element_type=jnp.float32)
                           + jnp.dot(g, uz_ref[...], preferred_element_type=jnp.float32))
        r = jax.nn.sigmoid(jnp.dot(ctx, wr_ref[...], preferred_element_type=jnp.float32)
                           + jnp.dot(g, ur_ref[...], preferred_element_type=jnp.float32))
        n = jnp.tanh(jnp.dot(ctx, wn_ref[...], preferred_element_type=jnp.float32)
                     + r * jnp.dot(g, un_ref[...], preferred_element_type=jnp.float32))
        g = (1.0 - z) * n + z * g
    g_out[...] = g


def _compute_mol(h, ids2d, wm_top, wm_bot_row, W_msg_m, Wz, Uz, Wr, Ur, Wn, Un):
    return pl.pallas_call(
        _mol_body,
        in_specs=[pl.BlockSpec((_N, _D), lambda: (0, 0)),
                  pl.BlockSpec((1, _N), lambda: (0, 0)),
                  pl.BlockSpec((_D, 1), lambda: (0, 0)),
                  pl.BlockSpec((1, _D), lambda: (0, 0)),
                  pl.BlockSpec((_D, _D), lambda: (0, 0)),
                  pl.BlockSpec((_D, _D), lambda: (0, 0)),
                  pl.BlockSpec((_D, _D), lambda: (0, 0)),
                  pl.BlockSpec((_D, _D), lambda: (0, 0)),
                  pl.BlockSpec((_D, _D), lambda: (0, 0)),
                  pl.BlockSpec((_D, _D), lambda: (0, 0)),
                  pl.BlockSpec((_D, _D), lambda: (0, 0))],
        out_specs=pl.BlockSpec((_G, _D), lambda: (0, 0)),
        out_shape=jax.ShapeDtypeStruct((_G, _D), jnp.float32),
    )(h, ids2d, wm_top, wm_bot_row, W_msg_m, Wz, Uz, Wr, Ur, Wn, Un)


def kernel(frag_node, frag_edge, edge_index, graph_ids, W_init, W_edge, w_att, W_msg,
           Wz_a, Uz_a, Wr_a, Ur_a, Wn_a, Un_a,
           w_att_m, W_msg_m, Wz_m, Uz_m, Wr_m, Ur_m, Wn_m, Un_m):
    src = edge_index[0]
    dst = edge_index[1]
    perm = []
    for kk in range(_D // 32):
        perm += list(range(32 * kk, 32 * kk + 32, 2))
        perm += list(range(32 * kk + 1, 32 * kk + 32, 2))
    perm = jnp.array(perm, jnp.int32)
    We_top = W_edge[:_D]
    wa_top = w_att[:_D]
    wa_bot = w_att[_D:, 0][perm]
    W_msg_p = W_msg[perm]
    W_bot = W_edge[_D:]
    Ebi = _compute_eb(frag_edge, W_bot[:, 0::2], W_bot[:, 1::2])
    h, Hn, hd = _compute_init(frag_node, W_init, We_top[:, 0::2],
                              We_top[:, 1::2], wa_top)
    for _ in range(_L):
        U2, sigf = _edge_kernel(src, dst, Hn, hd[:, 0], Ebi, wa_bot)
        h, Hn, hd = _compute_update(U2, sigf.reshape(_NC, _N).T, h, W_msg_p,
                                    Wz_a, Uz_a, Wr_a, Ur_a, Wn_a, Un_a,
                                    We_top[:, 0::2], We_top[:, 1::2], wa_top)
    g = _compute_mol(h, graph_ids.reshape(1, _N), w_att_m[:_D],
                     w_att_m[_D:, 0].reshape(1, _D), W_msg_m,
                     Wz_m, Uz_m, Wr_m, Ur_m, Wn_m, Un_m)
    return g
